# baseline (speedup 1.0000x reference)
import jax
import jax.numpy as jnp
from jax import lax
from jax.experimental import pallas as pl
from jax.experimental.pallas import tpu as pltpu

DEPTH = 2
DIM_EMB = 128
N_MODULES = 8


def _argmin8(score):
    k = score.shape[-1]
    min_s = jnp.min(score)
    iota = lax.broadcasted_iota(jnp.int32, score.shape, 1)
    return jnp.min(jnp.where(score == min_s, iota, k))


EXAMPLES_PER_STEP = 4


def _fused_kernel(x_ref, wctl_ref, bctl_ref, emb_ref, embc_ref,
                  wcomp_ref, bcomp_ref, y_ref, ctl_ref, ctln_ref):
    eps = EXAMPLES_PER_STEP
    hw = x_ref.shape[2]
    e2 = jnp.sum(emb_ref[...] ** 2, axis=0, keepdims=True)

    xms = [jnp.sum(x_ref[i], axis=1, keepdims=True) * (1.0 / hw)
           for i in range(eps)]
    xm = jnp.concatenate(xms, axis=1)

    ctl_all = jnp.dot(wctl_ref[...], xm,
                      preferred_element_type=jnp.float32) + bctl_ref[...]

    idxs = []
    for t in range(DEPTH):
        ctl_t = ctl_all[t * DIM_EMB:(t + 1) * DIM_EMB, :]
        dots = lax.dot_general(ctl_t, emb_ref[...], (((0,), (0,)), ((), ())),
                               preferred_element_type=jnp.float32)
        score = e2 - 2.0 * dots
        min_s = jnp.min(score, axis=1, keepdims=True)
        iota = lax.broadcasted_iota(jnp.int32, score.shape, 1)
        idxs.append(jnp.min(jnp.where(score == min_s, iota, score.shape[1]),
                            axis=1))

    for i in range(eps):
        ctl_ref[i, :, 0:1] = ctl_all[0 * DIM_EMB:1 * DIM_EMB, i:i + 1]
        ctl_ref[i, :, 1:2] = ctl_all[1 * DIM_EMB:2 * DIM_EMB, i:i + 1]

    composed = []
    for i in range(eps):
        idx0 = idxs[0][i]
        idx1 = idxs[1][i]
        ctln_ref[i, :, 0:1] = embc_ref[idx0]
        ctln_ref[i, :, 1:2] = embc_ref[idx1]
        w1 = wcomp_ref[idx0]
        w2 = wcomp_ref[idx1]
        b1 = bcomp_ref[idx0]
        b2 = bcomp_ref[idx1]
        w_eff = jnp.dot(w2, w1, preferred_element_type=jnp.float32)
        b_eff = jnp.dot(w2, b1, preferred_element_type=jnp.float32) + b2
        composed.append((w_eff.astype(jnp.bfloat16), b_eff))

    for i in range(eps):
        w_eff, b_eff = composed[i]
        y = jnp.dot(w_eff, x_ref[i].astype(jnp.bfloat16),
                    preferred_element_type=jnp.float32)
        y_ref[i] = (y + b_eff).astype(jnp.bfloat16)


def kernel(x, W_ctl, b_ctl, emb, W_comp, b_comp):
    Bn, C, H, W = x.shape
    HW = H * W
    W_ctl_dm = (W_ctl.reshape(DIM_EMB, DEPTH, C)
                .transpose(1, 0, 2).reshape(DEPTH * DIM_EMB, C))
    b_ctl_dm = b_ctl.reshape(DIM_EMB, DEPTH).T.reshape(DEPTH * DIM_EMB, 1)
    emb_cols = emb.T.reshape(N_MODULES, DIM_EMB, 1)
    b_comp_c = b_comp.reshape(N_MODULES, C, 1)

    eps = EXAMPLES_PER_STEP
    chunk = pl.pallas_call(
        _fused_kernel,
        grid=(1,),
        in_specs=[
            pl.BlockSpec((eps, C, HW), lambda e: (0, 0, 0)),
            pl.BlockSpec((DEPTH * DIM_EMB, C), lambda e: (0, 0)),
            pl.BlockSpec((DEPTH * DIM_EMB, 1), lambda e: (0, 0)),
            pl.BlockSpec((DIM_EMB, N_MODULES), lambda e: (0, 0)),
            pl.BlockSpec((N_MODULES, DIM_EMB, 1), lambda e: (0, 0, 0)),
            pl.BlockSpec((N_MODULES, C, C), lambda e: (0, 0, 0)),
            pl.BlockSpec((N_MODULES, C, 1), lambda e: (0, 0, 0)),
        ],
        out_specs=[
            pl.BlockSpec((eps, C, HW), lambda e: (0, 0, 0)),
            pl.BlockSpec((eps, DIM_EMB, DEPTH), lambda e: (0, 0, 0)),
            pl.BlockSpec((eps, DIM_EMB, DEPTH), lambda e: (0, 0, 0)),
        ],
        out_shape=[
            jax.ShapeDtypeStruct((eps, C, HW), jnp.bfloat16),
            jax.ShapeDtypeStruct((eps, DIM_EMB, DEPTH), jnp.float32),
            jax.ShapeDtypeStruct((eps, DIM_EMB, DEPTH), jnp.float32),
        ],
        compiler_params=pltpu.CompilerParams(
            dimension_semantics=("arbitrary",),
        ),
    )
    ys, ctls, ctlns = [], [], []
    for k in range(0, Bn, eps):
        x2_k = x[k:k + eps].reshape(eps, C, HW)
        y_k, ctl_k, ctln_k = chunk(x2_k, W_ctl_dm, b_ctl_dm, emb, emb_cols,
                                   W_comp, b_comp_c)
        ys.append(y_k.astype(jnp.float32).reshape(eps, C, H, W))
        ctls.append(ctl_k)
        ctlns.append(ctln_k)
    return (jnp.concatenate(ys, axis=0), jnp.concatenate(ctls, axis=0),
            jnp.concatenate(ctlns, axis=0))


# speedup vs baseline: 1.6840x; 1.6840x over previous
import jax
import jax.numpy as jnp
from jax import lax
from jax.experimental import pallas as pl
from jax.experimental.pallas import tpu as pltpu

DEPTH = 2
DIM_EMB = 128
N_MODULES = 8


def _argmin8(score):
    k = score.shape[-1]
    min_s = jnp.min(score)
    iota = lax.broadcasted_iota(jnp.int32, score.shape, 1)
    return jnp.min(jnp.where(score == min_s, iota, k))


EXAMPLES_PER_STEP = 8


def _fused_kernel(x_ref, wctl_ref, bctl_ref, emb_ref, embc_ref,
                  wcomp_ref, bcomp_ref, y_ref, ctl_ref, ctln_ref):
    eps = EXAMPLES_PER_STEP
    hw = x_ref.shape[2]
    e2 = jnp.sum(emb_ref[...] ** 2, axis=0, keepdims=True)

    xms = [jnp.sum(x_ref[i], axis=1, keepdims=True) * (1.0 / hw)
           for i in range(eps)]
    xm = jnp.concatenate(xms, axis=1)

    ctl_all = jnp.dot(wctl_ref[...], xm,
                      preferred_element_type=jnp.float32) + bctl_ref[...]

    idxs = []
    for t in range(DEPTH):
        ctl_t = ctl_all[t * DIM_EMB:(t + 1) * DIM_EMB, :]
        dots = lax.dot_general(ctl_t, emb_ref[...], (((0,), (0,)), ((), ())),
                               preferred_element_type=jnp.float32)
        score = e2 - 2.0 * dots
        min_s = jnp.min(score, axis=1, keepdims=True)
        iota = lax.broadcasted_iota(jnp.int32, score.shape, 1)
        idxs.append(jnp.min(jnp.where(score == min_s, iota, score.shape[1]),
                            axis=1))

    for i in range(eps):
        ctl_ref[i, :, 0:1] = ctl_all[0 * DIM_EMB:1 * DIM_EMB, i:i + 1]
        ctl_ref[i, :, 1:2] = ctl_all[1 * DIM_EMB:2 * DIM_EMB, i:i + 1]

    composed = []
    for i in range(eps):
        idx0 = idxs[0][i]
        idx1 = idxs[1][i]
        ctln_ref[i, :, 0:1] = embc_ref[idx0]
        ctln_ref[i, :, 1:2] = embc_ref[idx1]
        w1 = wcomp_ref[idx0]
        w2 = wcomp_ref[idx1]
        b1 = bcomp_ref[idx0]
        b2 = bcomp_ref[idx1]
        w_eff = jnp.dot(w2, w1, preferred_element_type=jnp.float32)
        b_eff = jnp.dot(w2, b1, preferred_element_type=jnp.float32) + b2
        composed.append((w_eff.astype(jnp.bfloat16), b_eff))

    for i in range(eps):
        w_eff, b_eff = composed[i]
        y = jnp.dot(w_eff, x_ref[i].astype(jnp.bfloat16),
                    preferred_element_type=jnp.float32)
        y_ref[i] = (y + b_eff).astype(jnp.bfloat16)


def kernel(x, W_ctl, b_ctl, emb, W_comp, b_comp):
    Bn, C, H, W = x.shape
    HW = H * W
    x2 = x.reshape(Bn, C, HW)
    W_ctl_dm = (W_ctl.reshape(DIM_EMB, DEPTH, C)
                .transpose(1, 0, 2).reshape(DEPTH * DIM_EMB, C))
    b_ctl_dm = b_ctl.reshape(DIM_EMB, DEPTH).T.reshape(DEPTH * DIM_EMB, 1)
    emb_cols = emb.T.reshape(N_MODULES, DIM_EMB, 1)
    b_comp_c = b_comp.reshape(N_MODULES, C, 1)

    eps = EXAMPLES_PER_STEP
    y, ctl, ctln = pl.pallas_call(
        _fused_kernel,
        grid=(Bn // eps,),
        in_specs=[
            pl.BlockSpec((eps, C, HW), lambda e: (e, 0, 0)),
            pl.BlockSpec((DEPTH * DIM_EMB, C), lambda e: (0, 0)),
            pl.BlockSpec((DEPTH * DIM_EMB, 1), lambda e: (0, 0)),
            pl.BlockSpec((DIM_EMB, N_MODULES), lambda e: (0, 0)),
            pl.BlockSpec((N_MODULES, DIM_EMB, 1), lambda e: (0, 0, 0)),
            pl.BlockSpec((N_MODULES, C, C), lambda e: (0, 0, 0)),
            pl.BlockSpec((N_MODULES, C, 1), lambda e: (0, 0, 0)),
        ],
        out_specs=[
            pl.BlockSpec((eps, C, HW), lambda e: (e, 0, 0)),
            pl.BlockSpec((eps, DIM_EMB, DEPTH), lambda e: (e, 0, 0)),
            pl.BlockSpec((eps, DIM_EMB, DEPTH), lambda e: (e, 0, 0)),
        ],
        out_shape=[
            jax.ShapeDtypeStruct((Bn, C, HW), jnp.bfloat16),
            jax.ShapeDtypeStruct((Bn, DIM_EMB, DEPTH), jnp.float32),
            jax.ShapeDtypeStruct((Bn, DIM_EMB, DEPTH), jnp.float32),
        ],
        compiler_params=pltpu.CompilerParams(
            dimension_semantics=("arbitrary",),
        ),
    )(x2, W_ctl_dm, b_ctl_dm, emb, emb_cols, W_comp, b_comp_c)
    return (y.astype(jnp.float32).reshape(Bn, C, H, W), ctl, ctln)
